# baseline (speedup 1.0000x reference)
import dataclasses
import functools

import jax
import jax.numpy as jnp
import numpy as np
from jax import lax
from jax.experimental import pallas as pl
from jax.experimental.pallas import tpu as pltpu
from jax.experimental.pallas import tpu_sc as plsc

N_NODES = 8192
FR = 128
LH = 256
N_ANGLES = 130816
N_MOL = 512
NP = 131072
SLICES = (8192, 57344, 57344, 8192)
NW = 32

CH = 64
NB = 3
TB = 2048


def _sc_compiler_params():
  cp = pltpu.CompilerParams()
  if "needs_layout_passes" in pltpu.CompilerParams.__dataclass_fields__:
    cp = dataclasses.replace(cp, needs_layout_passes=False)
  return cp


def _sc_gather_slice(r, xyzt, i0, i1, i2, sl):
  SPW = sl // NW
  NCH = SPW // CH
  mesh = plsc.VectorSubcoreMesh(core_axis_name="c", subcore_axis_name="s")
  f32 = jnp.float32
  out_type = (
      jax.ShapeDtypeStruct((sl, FR), f32),
      jax.ShapeDtypeStruct((sl, FR), f32),
      jax.ShapeDtypeStruct((sl, FR), f32),
      jax.ShapeDtypeStruct((sl,), f32),
      jax.ShapeDtypeStruct((sl,), f32),
      jax.ShapeDtypeStruct((sl,), f32),
  )
  scratch_types = [
      pltpu.VMEM((SPW,), jnp.int32),
      pltpu.VMEM((SPW,), jnp.int32),
      pltpu.VMEM((SPW,), jnp.int32),
      pltpu.VMEM((NB, CH, FR), f32),
      pltpu.VMEM((NB, CH, FR), f32),
      pltpu.VMEM((NB, CH, FR), f32),
      pltpu.VMEM((N_NODES,), f32),
      pltpu.VMEM((N_NODES,), f32),
      pltpu.VMEM((N_NODES,), f32),
      pltpu.VMEM((SPW,), f32),
      pltpu.VMEM((SPW,), f32),
      pltpu.VMEM((SPW,), f32),
      pltpu.SemaphoreType.DMA,
      pltpu.SemaphoreType.DMA,
      pltpu.SemaphoreType.DMA,
      pltpu.SemaphoreType.DMA,
      pltpu.SemaphoreType.DMA,
      pltpu.SemaphoreType.DMA,
      pltpu.SemaphoreType.DMA,
  ]

  @functools.partial(pl.kernel, out_type=out_type, mesh=mesh,
                     scratch_types=scratch_types,
                     compiler_params=_sc_compiler_params())
  def k(r_hbm, x_hbm, i0_hbm, i1_hbm, i2_hbm, g0, g1, g2, dh, n1h, n2h,
        iv0, iv1, iv2, b0, b1, b2, xs, ys, zs, dv, n1v, n2v,
        sg0, sg1, sg2, sw0, sw1, sw2, sx):
    wid = lax.axis_index("s") * 2 + lax.axis_index("c")
    base = wid * SPW
    pltpu.sync_copy(i0_hbm.at[pl.ds(base, SPW)], iv0)
    pltpu.sync_copy(i1_hbm.at[pl.ds(base, SPW)], iv1)
    pltpu.sync_copy(i2_hbm.at[pl.ds(base, SPW)], iv2)
    xcps = (
        pltpu.async_copy(x_hbm.at[pl.ds(0, N_NODES)], xs, sx),
        pltpu.async_copy(x_hbm.at[pl.ds(N_NODES, N_NODES)], ys, sx),
        pltpu.async_copy(x_hbm.at[pl.ds(2 * N_NODES, N_NODES)], zs, sx),
    )

    def geometry(c):
      for kk in range(CH // 16):
        o = c * CH + 16 * kk
        j0 = iv0[pl.ds(o, 16)]
        j1 = iv1[pl.ds(o, 16)]
        j2 = iv2[pl.ds(o, 16)]
        x0 = plsc.load_gather(xs, [j0])
        x1 = plsc.load_gather(xs, [j1])
        x2 = plsc.load_gather(xs, [j2])
        y0 = plsc.load_gather(ys, [j0])
        y1 = plsc.load_gather(ys, [j1])
        y2 = plsc.load_gather(ys, [j2])
        z0 = plsc.load_gather(zs, [j0])
        z1 = plsc.load_gather(zs, [j1])
        z2 = plsc.load_gather(zs, [j2])
        d1x = x1 - x0
        d1y = y1 - y0
        d1z = z1 - z0
        d2x = x2 - x1
        d2y = y2 - y1
        d2z = z2 - z1
        dv[pl.ds(o, 16)] = -(d1x * d2x + d1y * d2y + d1z * d2z)
        n1v[pl.ds(o, 16)] = d1x * d1x + d1y * d1y + d1z * d1z
        n2v[pl.ds(o, 16)] = d2x * d2x + d2y * d2y + d2z * d2z

    sg = (sg0, sg1, sg2)
    sw = (sw0, sw1, sw2)

    def fire_gather(c, b):
      off = c * CH
      return (
          pltpu.async_copy(r_hbm.at[iv0.at[pl.ds(off, CH)]], b0.at[b], sg[b]),
          pltpu.async_copy(r_hbm.at[iv1.at[pl.ds(off, CH)]], b1.at[b], sg[b]),
          pltpu.async_copy(r_hbm.at[iv2.at[pl.ds(off, CH)]], b2.at[b], sg[b]),
      )

    def fire_write(c, b):
      off = base + c * CH
      return (
          pltpu.async_copy(b0.at[b], g0.at[pl.ds(off, CH)], sw[b]),
          pltpu.async_copy(b1.at[b], g1.at[pl.ds(off, CH)], sw[b]),
          pltpu.async_copy(b2.at[b], g2.at[pl.ds(off, CH)], sw[b]),
      )

    gcps = {0: fire_gather(0, 0), 1: fire_gather(1, 1)}
    wcps = {}
    for c in range(NCH):
      b = c % NB
      if c + 2 < NCH:
        if c >= 1:
          for cp in wcps.pop(c - 1):
            cp.wait()
        gcps[c + 2] = fire_gather(c + 2, (c + 2) % NB)
      if c == 0:
        for cp in xcps:
          cp.wait()
      geometry(c)
      for cp in gcps.pop(c):
        cp.wait()
      wcps[c] = fire_write(c, b)
    for c, cps in wcps.items():
      for cp in cps:
        cp.wait()

    pltpu.sync_copy(dv, dh.at[pl.ds(base, SPW)])
    pltpu.sync_copy(n1v, n1h.at[pl.ds(base, SPW)])
    pltpu.sync_copy(n2v, n2h.at[pl.ds(base, SPW)])

  return k(r, xyzt, i0, i1, i2)


def _arccos(x):
  ax = jnp.abs(x)
  p = jnp.float32(-0.0012624911)
  p = p * ax + jnp.float32(0.0066700901)
  p = p * ax + jnp.float32(-0.0170881256)
  p = p * ax + jnp.float32(0.0308918810)
  p = p * ax + jnp.float32(-0.0501743046)
  p = p * ax + jnp.float32(0.0889789874)
  p = p * ax + jnp.float32(-0.2145988016)
  p = p * ax + jnp.float32(1.5707963050)
  r = jnp.sqrt(jnp.maximum(1.0 - ax, 0.0)) * p
  return jnp.where(x >= 0, r, jnp.float32(np.pi) - r)


WIN = 256
NMP = N_MOL + 128


def _tc_body(g0_r, g1_r, g2_r, d_r, n1_r, n2_r, w1_r, b1_r, w2_r,
             lo_r, hi_r, c_r, rb_r, wb_r, acc_r, out_r):
  i = pl.program_id(0)
  x1 = (g0_r[...] + g2_r[...]).astype(jnp.bfloat16)
  x2 = g1_r[...].astype(jnp.bfloat16)
  x = jnp.concatenate([x1, x2], axis=1)
  h = jnp.dot(x, w1_r[...], preferred_element_type=jnp.float32)
  h = jnp.tanh(h + b1_r[...])
  m = h * w2_r[...]
  t0m = jnp.sum(m[:, :LH], axis=1, keepdims=True)
  km = jnp.sum(m[:, LH:], axis=1, keepdims=True)
  t0l = jnp.transpose(t0m)
  kml = jnp.transpose(km)

  dot = d_r[...]
  n1 = n1_r[...]
  n2 = n2_r[...]
  cos = dot * lax.rsqrt(n1 * n2)
  theta = _arccos(cos * jnp.float32(1.0 / 1.000001))
  t0h = (c_r[0] + t0l) ** 2
  kh = (c_r[1] + kml) ** 2
  e = 0.5 * kh * (theta - t0h) ** 2

  wb = pl.multiple_of(wb_r[i], 128)
  shift = rb_r[0] + i * TB
  low = lo_r[0:1, pl.ds(wb, WIN)] - shift
  high = hi_r[0:1, pl.ds(wb, WIN)] - shift
  rows = lax.broadcasted_iota(jnp.int32, (TB, WIN), 0)
  oh = (rows >= low) & (rows < high)
  maskf = jnp.where(oh, jnp.float32(1.0), jnp.float32(0.0))
  part = lax.dot_general(e, maskf, (((1,), (0,)), ((), ())),
                         preferred_element_type=jnp.float32)

  @pl.when(i == 0)
  def _():
    out_r[...] = acc_r[...]

  out_r[0:1, pl.ds(wb, WIN)] += part


def _tc_compute(g0, g1, g2, d, n1, n2, w1c, b1c, w2r, lo, hi, consts, rb, wb,
                acc, sl):
  grid = (sl // TB,)
  full = lambda i: (0, 0)
  blk = lambda i: (i, 0)
  lane = lambda i: (0, i)
  return pl.pallas_call(
      _tc_body,
      grid=grid,
      in_specs=[pl.BlockSpec((TB, FR), blk)] * 3
      + [pl.BlockSpec((1, TB), lane)] * 3
      + [
          pl.BlockSpec((2 * FR, 2 * LH), full),
          pl.BlockSpec((1, 2 * LH), full),
          pl.BlockSpec((1, 2 * LH), full),
          pl.BlockSpec((1, NMP), full),
          pl.BlockSpec((1, NMP), full),
          pl.BlockSpec(memory_space=pltpu.SMEM),
          pl.BlockSpec(memory_space=pltpu.SMEM),
          pl.BlockSpec(memory_space=pltpu.SMEM),
          pl.BlockSpec((1, NMP), full),
      ],
      out_specs=pl.BlockSpec((1, NMP), full),
      out_shape=jax.ShapeDtypeStruct((1, NMP), jnp.float32),
  )(g0, g1, g2, d, n1, n2, w1c, b1c, w2r, lo, hi, consts, rb, wb, acc)


def kernel(r, xyz, t0_W1, t0_b1, t0_W2, t0_b2, k_W1, k_b1, k_W2, k_b2,
           angles, num_angles):
  f32 = jnp.float32
  angles = angles.astype(jnp.int32)
  pad_tri = jnp.array([[0, 1, 2]], dtype=jnp.int32)
  pad = jnp.broadcast_to(pad_tri, (NP - N_ANGLES, 3))
  ap = jnp.concatenate([angles, pad], axis=0)
  i0 = ap[:, 0]
  i1 = ap[:, 1]
  i2 = ap[:, 2]

  xyzt = xyz.astype(f32).T.reshape(3 * N_NODES)
  rf = r.astype(f32)

  w1c = jnp.concatenate([t0_W1, k_W1], axis=1).astype(jnp.bfloat16)
  b1c = jnp.concatenate([t0_b1, k_b1]).reshape(1, 2 * LH)
  w2r = jnp.concatenate([t0_W2[:, 0], k_W2[:, 0]]).reshape(1, 2 * LH)

  ends = jnp.cumsum(num_angles.astype(jnp.int32))
  starts = ends - num_angles.astype(jnp.int32)
  lo = jnp.pad(starts, (0, NMP - N_MOL),
               constant_values=np.int32(NP + 1)).reshape(1, NMP)
  hi = jnp.pad(ends, (0, NMP - N_MOL),
               constant_values=np.int32(0)).reshape(1, NMP)
  blk_rows = jnp.arange(NP // TB, dtype=jnp.int32) * TB
  base_mol = jnp.searchsorted(ends, blk_rows, side="right").astype(jnp.int32)
  wb_all = jnp.minimum(base_mol // 128 * 128, np.int32(N_MOL - 128))

  c0 = np.float32((109.5 * np.pi / 180.0) ** 0.5)
  c1 = np.float32(10.0 ** 0.5)

  consts = jnp.stack([c0 + t0_b2[0], c1 + k_b2[0]]).astype(f32)
  acc = jnp.zeros((1, NMP), f32)
  off = 0
  for sln in SLICES:
    sel = slice(off, off + sln)
    g0, g1, g2, d, n1, n2 = _sc_gather_slice(rf, xyzt,
                                             i0[sel], i1[sel], i2[sel], sln)
    rb = jnp.array([off], dtype=jnp.int32)
    wb = wb_all[off // TB:(off + sln) // TB]
    acc = _tc_compute(g0, g1, g2, d.reshape(1, sln), n1.reshape(1, sln),
                      n2.reshape(1, sln), w1c, b1c, w2r, lo, hi, consts,
                      rb, wb, acc, sln)
    off += sln
  return acc[:, :N_MOL].reshape(N_MOL, 1)


# speedup vs baseline: 1.0213x; 1.0213x over previous
import dataclasses
import functools

import jax
import jax.numpy as jnp
import numpy as np
from jax import lax
from jax.experimental import pallas as pl
from jax.experimental.pallas import tpu as pltpu
from jax.experimental.pallas import tpu_sc as plsc

N_NODES = 8192
FR = 128
LH = 256
N_ANGLES = 130816
N_MOL = 512
NP = 131072
SLICES = (16384, 32768, 32768, 32768, 16384)
NW = 32

CH = 64
NB = 3
TB = 2048


def _sc_compiler_params():
  cp = pltpu.CompilerParams()
  if "needs_layout_passes" in pltpu.CompilerParams.__dataclass_fields__:
    cp = dataclasses.replace(cp, needs_layout_passes=False)
  return cp


def _sc_gather_slice(r, xyzt, i0, i1, i2, sl):
  SPW = sl // NW
  NCH = SPW // CH
  mesh = plsc.VectorSubcoreMesh(core_axis_name="c", subcore_axis_name="s")
  f32 = jnp.float32
  out_type = (
      jax.ShapeDtypeStruct((sl, FR), f32),
      jax.ShapeDtypeStruct((sl, FR), f32),
      jax.ShapeDtypeStruct((sl, FR), f32),
      jax.ShapeDtypeStruct((sl,), f32),
      jax.ShapeDtypeStruct((sl,), f32),
      jax.ShapeDtypeStruct((sl,), f32),
  )
  scratch_types = [
      pltpu.VMEM((SPW,), jnp.int32),
      pltpu.VMEM((SPW,), jnp.int32),
      pltpu.VMEM((SPW,), jnp.int32),
      pltpu.VMEM((NB, CH, FR), f32),
      pltpu.VMEM((NB, CH, FR), f32),
      pltpu.VMEM((NB, CH, FR), f32),
      pltpu.VMEM((N_NODES,), f32),
      pltpu.VMEM((N_NODES,), f32),
      pltpu.VMEM((N_NODES,), f32),
      pltpu.VMEM((SPW,), f32),
      pltpu.VMEM((SPW,), f32),
      pltpu.VMEM((SPW,), f32),
      pltpu.SemaphoreType.DMA,
      pltpu.SemaphoreType.DMA,
      pltpu.SemaphoreType.DMA,
      pltpu.SemaphoreType.DMA,
      pltpu.SemaphoreType.DMA,
      pltpu.SemaphoreType.DMA,
      pltpu.SemaphoreType.DMA,
  ]

  @functools.partial(pl.kernel, out_type=out_type, mesh=mesh,
                     scratch_types=scratch_types,
                     compiler_params=_sc_compiler_params())
  def k(r_hbm, x_hbm, i0_hbm, i1_hbm, i2_hbm, g0, g1, g2, dh, n1h, n2h,
        iv0, iv1, iv2, b0, b1, b2, xs, ys, zs, dv, n1v, n2v,
        sg0, sg1, sg2, sw0, sw1, sw2, sx):
    wid = lax.axis_index("s") * 2 + lax.axis_index("c")
    base = wid * SPW
    pltpu.sync_copy(i0_hbm.at[pl.ds(base, SPW)], iv0)
    pltpu.sync_copy(i1_hbm.at[pl.ds(base, SPW)], iv1)
    pltpu.sync_copy(i2_hbm.at[pl.ds(base, SPW)], iv2)
    xcps = (
        pltpu.async_copy(x_hbm.at[pl.ds(0, N_NODES)], xs, sx),
        pltpu.async_copy(x_hbm.at[pl.ds(N_NODES, N_NODES)], ys, sx),
        pltpu.async_copy(x_hbm.at[pl.ds(2 * N_NODES, N_NODES)], zs, sx),
    )

    def geometry(c):
      for kk in range(CH // 16):
        o = c * CH + 16 * kk
        j0 = iv0[pl.ds(o, 16)]
        j1 = iv1[pl.ds(o, 16)]
        j2 = iv2[pl.ds(o, 16)]
        x0 = plsc.load_gather(xs, [j0])
        x1 = plsc.load_gather(xs, [j1])
        x2 = plsc.load_gather(xs, [j2])
        y0 = plsc.load_gather(ys, [j0])
        y1 = plsc.load_gather(ys, [j1])
        y2 = plsc.load_gather(ys, [j2])
        z0 = plsc.load_gather(zs, [j0])
        z1 = plsc.load_gather(zs, [j1])
        z2 = plsc.load_gather(zs, [j2])
        d1x = x1 - x0
        d1y = y1 - y0
        d1z = z1 - z0
        d2x = x2 - x1
        d2y = y2 - y1
        d2z = z2 - z1
        dv[pl.ds(o, 16)] = -(d1x * d2x + d1y * d2y + d1z * d2z)
        n1v[pl.ds(o, 16)] = d1x * d1x + d1y * d1y + d1z * d1z
        n2v[pl.ds(o, 16)] = d2x * d2x + d2y * d2y + d2z * d2z

    sg = (sg0, sg1, sg2)
    sw = (sw0, sw1, sw2)

    def fire_gather(c, b):
      off = c * CH
      return (
          pltpu.async_copy(r_hbm.at[iv0.at[pl.ds(off, CH)]], b0.at[b], sg[b]),
          pltpu.async_copy(r_hbm.at[iv1.at[pl.ds(off, CH)]], b1.at[b], sg[b]),
          pltpu.async_copy(r_hbm.at[iv2.at[pl.ds(off, CH)]], b2.at[b], sg[b]),
      )

    def fire_write(c, b):
      off = base + c * CH
      return (
          pltpu.async_copy(b0.at[b], g0.at[pl.ds(off, CH)], sw[b]),
          pltpu.async_copy(b1.at[b], g1.at[pl.ds(off, CH)], sw[b]),
          pltpu.async_copy(b2.at[b], g2.at[pl.ds(off, CH)], sw[b]),
      )

    gcps = {0: fire_gather(0, 0), 1: fire_gather(1, 1)}
    wcps = {}
    for c in range(NCH):
      b = c % NB
      if c + 2 < NCH:
        if c >= 1:
          for cp in wcps.pop(c - 1):
            cp.wait()
        gcps[c + 2] = fire_gather(c + 2, (c + 2) % NB)
      if c == 0:
        for cp in xcps:
          cp.wait()
      geometry(c)
      for cp in gcps.pop(c):
        cp.wait()
      wcps[c] = fire_write(c, b)
    for c, cps in wcps.items():
      for cp in cps:
        cp.wait()

    pltpu.sync_copy(dv, dh.at[pl.ds(base, SPW)])
    pltpu.sync_copy(n1v, n1h.at[pl.ds(base, SPW)])
    pltpu.sync_copy(n2v, n2h.at[pl.ds(base, SPW)])

  return k(r, xyzt, i0, i1, i2)


def _arccos(x):
  ax = jnp.abs(x)
  p = jnp.float32(-0.0012624911)
  p = p * ax + jnp.float32(0.0066700901)
  p = p * ax + jnp.float32(-0.0170881256)
  p = p * ax + jnp.float32(0.0308918810)
  p = p * ax + jnp.float32(-0.0501743046)
  p = p * ax + jnp.float32(0.0889789874)
  p = p * ax + jnp.float32(-0.2145988016)
  p = p * ax + jnp.float32(1.5707963050)
  r = jnp.sqrt(jnp.maximum(1.0 - ax, 0.0)) * p
  return jnp.where(x >= 0, r, jnp.float32(np.pi) - r)


WIN = 256
NMP = N_MOL + 128


def _tc_body(g0_r, g1_r, g2_r, d_r, n1_r, n2_r, w1_r, b1_r, w2_r,
             lo_r, hi_r, c_r, rb_r, wb_r, acc_r, out_r):
  i = pl.program_id(0)
  x1 = (g0_r[...] + g2_r[...]).astype(jnp.bfloat16)
  x2 = g1_r[...].astype(jnp.bfloat16)
  x = jnp.concatenate([x1, x2], axis=1)
  h = jnp.dot(x, w1_r[...], preferred_element_type=jnp.float32)
  h = jnp.tanh(h + b1_r[...])
  m = h * w2_r[...]
  t0m = jnp.sum(m[:, :LH], axis=1, keepdims=True)
  km = jnp.sum(m[:, LH:], axis=1, keepdims=True)
  t0l = jnp.transpose(t0m)
  kml = jnp.transpose(km)

  dot = d_r[...]
  n1 = n1_r[...]
  n2 = n2_r[...]
  cos = dot * lax.rsqrt(n1 * n2)
  theta = _arccos(cos * jnp.float32(1.0 / 1.000001))
  t0h = (c_r[0] + t0l) ** 2
  kh = (c_r[1] + kml) ** 2
  e = 0.5 * kh * (theta - t0h) ** 2

  wb = pl.multiple_of(wb_r[i], 128)
  shift = rb_r[0] + i * TB
  low = lo_r[0:1, pl.ds(wb, WIN)] - shift
  high = hi_r[0:1, pl.ds(wb, WIN)] - shift
  rows = lax.broadcasted_iota(jnp.int32, (TB, WIN), 0)
  oh = (rows >= low) & (rows < high)
  maskf = jnp.where(oh, jnp.float32(1.0), jnp.float32(0.0))
  part = lax.dot_general(e, maskf, (((1,), (0,)), ((), ())),
                         preferred_element_type=jnp.float32)

  @pl.when(i == 0)
  def _():
    out_r[...] = acc_r[...]

  out_r[0:1, pl.ds(wb, WIN)] += part


def _tc_compute(g0, g1, g2, d, n1, n2, w1c, b1c, w2r, lo, hi, consts, rb, wb,
                acc, sl):
  grid = (sl // TB,)
  full = lambda i: (0, 0)
  blk = lambda i: (i, 0)
  lane = lambda i: (0, i)
  return pl.pallas_call(
      _tc_body,
      grid=grid,
      in_specs=[pl.BlockSpec((TB, FR), blk)] * 3
      + [pl.BlockSpec((1, TB), lane)] * 3
      + [
          pl.BlockSpec((2 * FR, 2 * LH), full),
          pl.BlockSpec((1, 2 * LH), full),
          pl.BlockSpec((1, 2 * LH), full),
          pl.BlockSpec((1, NMP), full),
          pl.BlockSpec((1, NMP), full),
          pl.BlockSpec(memory_space=pltpu.SMEM),
          pl.BlockSpec(memory_space=pltpu.SMEM),
          pl.BlockSpec(memory_space=pltpu.SMEM),
          pl.BlockSpec((1, NMP), full),
      ],
      out_specs=pl.BlockSpec((1, NMP), full),
      out_shape=jax.ShapeDtypeStruct((1, NMP), jnp.float32),
  )(g0, g1, g2, d, n1, n2, w1c, b1c, w2r, lo, hi, consts, rb, wb, acc)


def kernel(r, xyz, t0_W1, t0_b1, t0_W2, t0_b2, k_W1, k_b1, k_W2, k_b2,
           angles, num_angles):
  f32 = jnp.float32
  angles = angles.astype(jnp.int32)
  pad_tri = jnp.array([[0, 1, 2]], dtype=jnp.int32)
  pad = jnp.broadcast_to(pad_tri, (NP - N_ANGLES, 3))
  ap = jnp.concatenate([angles, pad], axis=0)
  i0 = ap[:, 0]
  i1 = ap[:, 1]
  i2 = ap[:, 2]

  xyzt = xyz.astype(f32).T.reshape(3 * N_NODES)
  rf = r.astype(f32)

  w1c = jnp.concatenate([t0_W1, k_W1], axis=1).astype(jnp.bfloat16)
  b1c = jnp.concatenate([t0_b1, k_b1]).reshape(1, 2 * LH)
  w2r = jnp.concatenate([t0_W2[:, 0], k_W2[:, 0]]).reshape(1, 2 * LH)

  ends = jnp.cumsum(num_angles.astype(jnp.int32))
  starts = ends - num_angles.astype(jnp.int32)
  lo = jnp.pad(starts, (0, NMP - N_MOL),
               constant_values=np.int32(NP + 1)).reshape(1, NMP)
  hi = jnp.pad(ends, (0, NMP - N_MOL),
               constant_values=np.int32(0)).reshape(1, NMP)
  blk_rows = jnp.arange(NP // TB, dtype=jnp.int32) * TB
  base_mol = jnp.searchsorted(ends, blk_rows, side="right").astype(jnp.int32)
  wb_all = jnp.minimum(base_mol // 128 * 128, np.int32(N_MOL - 128))

  c0 = np.float32((109.5 * np.pi / 180.0) ** 0.5)
  c1 = np.float32(10.0 ** 0.5)

  consts = jnp.stack([c0 + t0_b2[0], c1 + k_b2[0]]).astype(f32)
  acc = jnp.zeros((1, NMP), f32)
  off = 0
  for sln in SLICES:
    sel = slice(off, off + sln)
    g0, g1, g2, d, n1, n2 = _sc_gather_slice(rf, xyzt,
                                             i0[sel], i1[sel], i2[sel], sln)
    rb = jnp.array([off], dtype=jnp.int32)
    wb = wb_all[off // TB:(off + sln) // TB]
    acc = _tc_compute(g0, g1, g2, d.reshape(1, sln), n1.reshape(1, sln),
                      n2.reshape(1, sln), w1c, b1c, w2r, lo, hi, consts,
                      rb, wb, acc, sln)
    off += sln
  return acc[:, :N_MOL].reshape(N_MOL, 1)


# speedup vs baseline: 1.0604x; 1.0382x over previous
import dataclasses
import functools

import jax
import jax.numpy as jnp
import numpy as np
from jax import lax
from jax.experimental import pallas as pl
from jax.experimental.pallas import tpu as pltpu
from jax.experimental.pallas import tpu_sc as plsc

N_NODES = 8192
FR = 128
LH = 256
N_ANGLES = 130816
N_MOL = 512
NP = 131072
SLICES = (32768, 32768, 32768, 32768)
NW = 32

CH = 64
NB = 3
TB = 2048


def _sc_compiler_params():
  cp = pltpu.CompilerParams()
  if "needs_layout_passes" in pltpu.CompilerParams.__dataclass_fields__:
    cp = dataclasses.replace(cp, needs_layout_passes=False)
  return cp


def _sc_gather_slice(r, xyzt, i0, i1, i2, sl):
  SPW = sl // NW
  NCH = SPW // CH
  mesh = plsc.VectorSubcoreMesh(core_axis_name="c", subcore_axis_name="s")
  f32 = jnp.float32
  out_type = (
      jax.ShapeDtypeStruct((sl, FR), f32),
      jax.ShapeDtypeStruct((sl, FR), f32),
      jax.ShapeDtypeStruct((sl, FR), f32),
      jax.ShapeDtypeStruct((sl,), f32),
      jax.ShapeDtypeStruct((sl,), f32),
      jax.ShapeDtypeStruct((sl,), f32),
  )
  scratch_types = [
      pltpu.VMEM((SPW,), jnp.int32),
      pltpu.VMEM((SPW,), jnp.int32),
      pltpu.VMEM((SPW,), jnp.int32),
      pltpu.VMEM((NB, CH, FR), f32),
      pltpu.VMEM((NB, CH, FR), f32),
      pltpu.VMEM((NB, CH, FR), f32),
      pltpu.VMEM((N_NODES,), f32),
      pltpu.VMEM((N_NODES,), f32),
      pltpu.VMEM((N_NODES,), f32),
      pltpu.VMEM((SPW,), f32),
      pltpu.VMEM((SPW,), f32),
      pltpu.VMEM((SPW,), f32),
      pltpu.SemaphoreType.DMA,
      pltpu.SemaphoreType.DMA,
      pltpu.SemaphoreType.DMA,
      pltpu.SemaphoreType.DMA,
      pltpu.SemaphoreType.DMA,
      pltpu.SemaphoreType.DMA,
      pltpu.SemaphoreType.DMA,
  ]

  @functools.partial(pl.kernel, out_type=out_type, mesh=mesh,
                     scratch_types=scratch_types,
                     compiler_params=_sc_compiler_params())
  def k(r_hbm, x_hbm, i0_hbm, i1_hbm, i2_hbm, g0, g1, g2, dh, n1h, n2h,
        iv0, iv1, iv2, b0, b1, b2, xs, ys, zs, dv, n1v, n2v,
        sg0, sg1, sg2, sw0, sw1, sw2, sx):
    wid = lax.axis_index("s") * 2 + lax.axis_index("c")
    base = wid * SPW
    pltpu.sync_copy(i0_hbm.at[pl.ds(base, SPW)], iv0)
    pltpu.sync_copy(i1_hbm.at[pl.ds(base, SPW)], iv1)
    pltpu.sync_copy(i2_hbm.at[pl.ds(base, SPW)], iv2)
    xcps = (
        pltpu.async_copy(x_hbm.at[pl.ds(0, N_NODES)], xs, sx),
        pltpu.async_copy(x_hbm.at[pl.ds(N_NODES, N_NODES)], ys, sx),
        pltpu.async_copy(x_hbm.at[pl.ds(2 * N_NODES, N_NODES)], zs, sx),
    )

    def geometry(c):
      for kk in range(CH // 16):
        o = c * CH + 16 * kk
        j0 = iv0[pl.ds(o, 16)]
        j1 = iv1[pl.ds(o, 16)]
        j2 = iv2[pl.ds(o, 16)]
        x0 = plsc.load_gather(xs, [j0])
        x1 = plsc.load_gather(xs, [j1])
        x2 = plsc.load_gather(xs, [j2])
        y0 = plsc.load_gather(ys, [j0])
        y1 = plsc.load_gather(ys, [j1])
        y2 = plsc.load_gather(ys, [j2])
        z0 = plsc.load_gather(zs, [j0])
        z1 = plsc.load_gather(zs, [j1])
        z2 = plsc.load_gather(zs, [j2])
        d1x = x1 - x0
        d1y = y1 - y0
        d1z = z1 - z0
        d2x = x2 - x1
        d2y = y2 - y1
        d2z = z2 - z1
        dv[pl.ds(o, 16)] = -(d1x * d2x + d1y * d2y + d1z * d2z)
        n1v[pl.ds(o, 16)] = d1x * d1x + d1y * d1y + d1z * d1z
        n2v[pl.ds(o, 16)] = d2x * d2x + d2y * d2y + d2z * d2z

    sg = (sg0, sg1, sg2)
    sw = (sw0, sw1, sw2)

    def fire_gather(c, b):
      off = c * CH
      return (
          pltpu.async_copy(r_hbm.at[iv0.at[pl.ds(off, CH)]], b0.at[b], sg[b]),
          pltpu.async_copy(r_hbm.at[iv1.at[pl.ds(off, CH)]], b1.at[b], sg[b]),
          pltpu.async_copy(r_hbm.at[iv2.at[pl.ds(off, CH)]], b2.at[b], sg[b]),
      )

    def fire_write(c, b):
      off = base + c * CH
      return (
          pltpu.async_copy(b0.at[b], g0.at[pl.ds(off, CH)], sw[b]),
          pltpu.async_copy(b1.at[b], g1.at[pl.ds(off, CH)], sw[b]),
          pltpu.async_copy(b2.at[b], g2.at[pl.ds(off, CH)], sw[b]),
      )

    gcps = {0: fire_gather(0, 0), 1: fire_gather(1, 1)}
    wcps = {}
    for c in range(NCH):
      b = c % NB
      if c + 2 < NCH:
        if c >= 1:
          for cp in wcps.pop(c - 1):
            cp.wait()
        gcps[c + 2] = fire_gather(c + 2, (c + 2) % NB)
      if c == 0:
        for cp in xcps:
          cp.wait()
      geometry(c)
      for cp in gcps.pop(c):
        cp.wait()
      wcps[c] = fire_write(c, b)
    for c, cps in wcps.items():
      for cp in cps:
        cp.wait()

    pltpu.sync_copy(dv, dh.at[pl.ds(base, SPW)])
    pltpu.sync_copy(n1v, n1h.at[pl.ds(base, SPW)])
    pltpu.sync_copy(n2v, n2h.at[pl.ds(base, SPW)])

  return k(r, xyzt, i0, i1, i2)


def _arccos(x):
  ax = jnp.abs(x)
  p = jnp.float32(-0.0012624911)
  p = p * ax + jnp.float32(0.0066700901)
  p = p * ax + jnp.float32(-0.0170881256)
  p = p * ax + jnp.float32(0.0308918810)
  p = p * ax + jnp.float32(-0.0501743046)
  p = p * ax + jnp.float32(0.0889789874)
  p = p * ax + jnp.float32(-0.2145988016)
  p = p * ax + jnp.float32(1.5707963050)
  r = jnp.sqrt(jnp.maximum(1.0 - ax, 0.0)) * p
  return jnp.where(x >= 0, r, jnp.float32(np.pi) - r)


WIN = 256
NMP = N_MOL + 128


def _tc_body(g0_r, g1_r, g2_r, d_r, n1_r, n2_r, w1_r, b1_r, w2_r,
             lo_r, hi_r, c_r, rb_r, wb_r, acc_r, out_r):
  i = pl.program_id(0)
  x1 = (g0_r[...] + g2_r[...]).astype(jnp.bfloat16)
  x2 = g1_r[...].astype(jnp.bfloat16)
  x = jnp.concatenate([x1, x2], axis=1)
  h = jnp.dot(x, w1_r[...], preferred_element_type=jnp.float32)
  h = jnp.tanh(h + b1_r[...])
  m = h * w2_r[...]
  t0m = jnp.sum(m[:, :LH], axis=1, keepdims=True)
  km = jnp.sum(m[:, LH:], axis=1, keepdims=True)
  t0l = jnp.transpose(t0m)
  kml = jnp.transpose(km)

  dot = d_r[...]
  n1 = n1_r[...]
  n2 = n2_r[...]
  cos = dot * lax.rsqrt(n1 * n2)
  theta = _arccos(cos * jnp.float32(1.0 / 1.000001))
  t0h = (c_r[0] + t0l) ** 2
  kh = (c_r[1] + kml) ** 2
  e = 0.5 * kh * (theta - t0h) ** 2

  wb = pl.multiple_of(wb_r[i], 128)
  shift = rb_r[0] + i * TB
  low = lo_r[0:1, pl.ds(wb, WIN)] - shift
  high = hi_r[0:1, pl.ds(wb, WIN)] - shift
  rows = lax.broadcasted_iota(jnp.int32, (TB, WIN), 0)
  oh = (rows >= low) & (rows < high)
  maskf = jnp.where(oh, jnp.float32(1.0), jnp.float32(0.0))
  part = lax.dot_general(e, maskf, (((1,), (0,)), ((), ())),
                         preferred_element_type=jnp.float32)

  @pl.when(i == 0)
  def _():
    out_r[...] = acc_r[...]

  out_r[0:1, pl.ds(wb, WIN)] += part


def _tc_compute(g0, g1, g2, d, n1, n2, w1c, b1c, w2r, lo, hi, consts, rb, wb,
                acc, sl):
  grid = (sl // TB,)
  full = lambda i: (0, 0)
  blk = lambda i: (i, 0)
  lane = lambda i: (0, i)
  return pl.pallas_call(
      _tc_body,
      grid=grid,
      in_specs=[pl.BlockSpec((TB, FR), blk)] * 3
      + [pl.BlockSpec((1, TB), lane)] * 3
      + [
          pl.BlockSpec((2 * FR, 2 * LH), full),
          pl.BlockSpec((1, 2 * LH), full),
          pl.BlockSpec((1, 2 * LH), full),
          pl.BlockSpec((1, NMP), full),
          pl.BlockSpec((1, NMP), full),
          pl.BlockSpec(memory_space=pltpu.SMEM),
          pl.BlockSpec(memory_space=pltpu.SMEM),
          pl.BlockSpec(memory_space=pltpu.SMEM),
          pl.BlockSpec((1, NMP), full),
      ],
      out_specs=pl.BlockSpec((1, NMP), full),
      out_shape=jax.ShapeDtypeStruct((1, NMP), jnp.float32),
  )(g0, g1, g2, d, n1, n2, w1c, b1c, w2r, lo, hi, consts, rb, wb, acc)


def kernel(r, xyz, t0_W1, t0_b1, t0_W2, t0_b2, k_W1, k_b1, k_W2, k_b2,
           angles, num_angles):
  f32 = jnp.float32
  angles = angles.astype(jnp.int32)
  pad_tri = jnp.array([[0, 1, 2]], dtype=jnp.int32)
  pad = jnp.broadcast_to(pad_tri, (NP - N_ANGLES, 3))
  ap = jnp.concatenate([angles, pad], axis=0)
  i0 = ap[:, 0]
  i1 = ap[:, 1]
  i2 = ap[:, 2]

  xyzt = xyz.astype(f32).T.reshape(3 * N_NODES)
  rf = r.astype(f32)

  w1c = jnp.concatenate([t0_W1, k_W1], axis=1).astype(jnp.bfloat16)
  b1c = jnp.concatenate([t0_b1, k_b1]).reshape(1, 2 * LH)
  w2r = jnp.concatenate([t0_W2[:, 0], k_W2[:, 0]]).reshape(1, 2 * LH)

  ends = jnp.cumsum(num_angles.astype(jnp.int32))
  starts = ends - num_angles.astype(jnp.int32)
  lo = jnp.pad(starts, (0, NMP - N_MOL),
               constant_values=np.int32(NP + 1)).reshape(1, NMP)
  hi = jnp.pad(ends, (0, NMP - N_MOL),
               constant_values=np.int32(0)).reshape(1, NMP)
  blk_rows = jnp.arange(NP // TB, dtype=jnp.int32) * TB
  base_mol = jnp.searchsorted(ends, blk_rows, side="right").astype(jnp.int32)
  wb_all = jnp.minimum(base_mol // 128 * 128, np.int32(N_MOL - 128))

  c0 = np.float32((109.5 * np.pi / 180.0) ** 0.5)
  c1 = np.float32(10.0 ** 0.5)

  consts = jnp.stack([c0 + t0_b2[0], c1 + k_b2[0]]).astype(f32)
  acc = jnp.zeros((1, NMP), f32)
  off = 0
  for sln in SLICES:
    sel = slice(off, off + sln)
    g0, g1, g2, d, n1, n2 = _sc_gather_slice(rf, xyzt,
                                             i0[sel], i1[sel], i2[sel], sln)
    rb = jnp.array([off], dtype=jnp.int32)
    wb = wb_all[off // TB:(off + sln) // TB]
    acc = _tc_compute(g0, g1, g2, d.reshape(1, sln), n1.reshape(1, sln),
                      n2.reshape(1, sln), w1c, b1c, w2r, lo, hi, consts,
                      rb, wb, acc, sln)
    off += sln
  return acc[:, :N_MOL].reshape(N_MOL, 1)
